# baseline (speedup 1.0000x reference)
import jax
import jax.numpy as jnp
from jax import lax
from jax.experimental import pallas as pl
from jax.experimental.pallas import tpu as pltpu
from jax.experimental.pallas import tpu_sc as plsc

NUM_CLASSES = 1000
FEAT_DIM = 128
BATCH = 16384
ALPHA = 0.99

PAD_CLASSES = 1024
NC = 2
NS = 16
NWT = NC * NS
ROWS_PER_W = BATCH // NWT
CHUNK = 128
NCHUNK = ROWS_PER_W // CHUNK
CLS_PER_S = PAD_CLASSES // NS
LANES = 16
VL = FEAT_DIM // LANES
CNT_ROWS = PAD_CLASSES // FEAT_DIM
CNT_BLK = 2 * CNT_ROWS


def _scatter_body(feat_hbm, lbl_hbm, lblf_hbm, psum_hbm, pcnt_hbm,
                  lbl_v, lblf_v, feat_v, cnt_v, cnt4_v, idx_v, zero_v,
                  shared_acc, shared_cnt,
                  sem0, sem1, sem2, sem3):
    cid = lax.axis_index("c")
    sid = lax.axis_index("s")
    wid = cid * NS + sid
    cls_base = sid * CLS_PER_S
    zeros16 = jnp.zeros((LANES,), jnp.float32)

    def zero_row(r, _):
        for j in range(VL):
            zero_v[r, pl.ds(j * LANES, LANES)] = zeros16
        return _
    lax.fori_loop(0, CLS_PER_S, zero_row, None)

    pltpu.sync_copy(zero_v, shared_acc.at[pl.ds(cls_base, CLS_PER_S)])

    @pl.when(sid == 0)
    def _():
        pltpu.sync_copy(zero_v.at[pl.ds(0, CNT_BLK)], shared_cnt)
    plsc.subcore_barrier()

    pltpu.sync_copy(lbl_hbm.at[wid], lbl_v)
    pltpu.sync_copy(lblf_hbm.at[wid], lblf_v)
    sems = [sem0, sem1, sem2, sem3]
    copies = []
    for j in range(NCHUNK):
        copies.append(pltpu.async_copy(
            feat_hbm.at[pl.ds(wid * ROWS_PER_W + j * CHUNK, CHUNK)],
            feat_v.at[j], sems[j]))

    idx_v[pl.ds(0, LANES)] = lax.iota(jnp.int32, LANES)

    def czero(r, _):
        for j in range(VL):
            cnt_v[r, pl.ds(j * LANES, LANES)] = zeros16
        return _
    lax.fori_loop(0, CNT_BLK, czero, None)

    def czero4(r, _):
        for k in range(4):
            for j in range(VL):
                cnt4_v[k, r, pl.ds(j * LANES, LANES)] = zeros16
        return _
    lax.fori_loop(0, CNT_ROWS, czero4, None)

    lane_iota = lax.iota(jnp.int32, LANES)

    def count16(i, _):
        lbl16 = lblf_v[i, pl.ds(0, LANES)]
        for u in range(LANES):
            l = lbl16[u]
            row = lax.shift_right_logical(l, 7)
            colg = l & 0x70
            lane = l & 0xF
            onehot = jnp.where(lane_iota == lane, 1.0, 0.0)
            k = u & 3
            cnt4_v[k, row, pl.ds(colg, LANES)] = (
                cnt4_v[k, row, pl.ds(colg, LANES)] + onehot)
        return _
    lax.fori_loop(0, ROWS_PER_W // LANES, count16, None)

    def cmerge(r, _):
        for j in range(VL):
            sl = pl.ds(j * LANES, LANES)
            cnt_v[r, sl] = ((cnt4_v[0, r, sl] + cnt4_v[1, r, sl]) +
                            (cnt4_v[2, r, sl] + cnt4_v[3, r, sl]))
        return _
    lax.fori_loop(0, CNT_ROWS, cmerge, None)

    for j in range(NCHUNK):
        copies[j].wait()
        pltpu.sync_copy(feat_v.at[j], shared_acc.at[lbl_v.at[j]], add=True)

    pltpu.sync_copy(cnt_v, shared_cnt.at[idx_v], add=True)
    plsc.subcore_barrier()

    pltpu.sync_copy(shared_acc.at[pl.ds(cls_base, CLS_PER_S)],
                    psum_hbm.at[cid, pl.ds(cls_base, CLS_PER_S)])

    @pl.when(sid == 0)
    def _():
        pltpu.sync_copy(shared_cnt.at[pl.ds(0, CNT_ROWS)],
                        pcnt_hbm.at[cid])


def _ema_body(psum_ref, c0_ref, c1_ref, proto_ref, out_ref):
    s = psum_ref[0, :NUM_CLASSES, :] + psum_ref[1, :NUM_CLASSES, :]
    c = (c0_ref[...] + c1_ref[...])[:NUM_CLASSES, :]
    cb = jnp.broadcast_to(c, (NUM_CLASSES, FEAT_DIM))
    p = proto_ref[...]
    mean = s / jnp.maximum(cb, 1.0)
    out_ref[...] = jnp.where(cb > 0.0, ALPHA * p + (1.0 - ALPHA) * mean, p)


@jax.jit
def _run(features, labels3, prototypes):
    mesh = plsc.VectorSubcoreMesh(
        core_axis_name="c", subcore_axis_name="s", num_cores=NC,
        num_subcores=NS)
    psum, pcnt = pl.kernel(
        _scatter_body,
        out_type=(
            jax.ShapeDtypeStruct((NC, PAD_CLASSES, FEAT_DIM), jnp.float32),
            jax.ShapeDtypeStruct((NC, CNT_ROWS, FEAT_DIM), jnp.float32)),
        mesh=mesh,
        scratch_types=[
            pltpu.VMEM((NCHUNK, CHUNK), jnp.int32),
            pltpu.VMEM((ROWS_PER_W // LANES, LANES), jnp.int32),
            pltpu.VMEM((NCHUNK, CHUNK, FEAT_DIM), jnp.float32),
            pltpu.VMEM((CNT_BLK, FEAT_DIM), jnp.float32),
            pltpu.VMEM((4, CNT_ROWS, FEAT_DIM), jnp.float32),
            pltpu.VMEM((LANES,), jnp.int32),
            pltpu.VMEM((CLS_PER_S, FEAT_DIM), jnp.float32),
            pltpu.VMEM_SHARED((PAD_CLASSES, FEAT_DIM), jnp.float32),
            pltpu.VMEM_SHARED((CNT_BLK, FEAT_DIM), jnp.float32),
            pltpu.SemaphoreType.DMA,
            pltpu.SemaphoreType.DMA,
            pltpu.SemaphoreType.DMA,
            pltpu.SemaphoreType.DMA,
        ],
    )(features, labels3, labels3.reshape(NWT, ROWS_PER_W // LANES, LANES))

    c0 = pcnt[0].reshape(PAD_CLASSES, 1)
    c1 = pcnt[1].reshape(PAD_CLASSES, 1)
    out = pl.pallas_call(
        _ema_body,
        out_shape=jax.ShapeDtypeStruct((NUM_CLASSES, FEAT_DIM), jnp.float32),
    )(psum, c0, c1, prototypes)
    return out


def kernel(features, labels, prototypes):
    labels3 = labels.astype(jnp.int32).reshape(NWT, NCHUNK, CHUNK)
    return _run(features, labels3, prototypes)


# speedup vs baseline: 1.0977x; 1.0977x over previous
import jax
import jax.numpy as jnp
from jax import lax
from jax.experimental import pallas as pl
from jax.experimental.pallas import tpu as pltpu
from jax.experimental.pallas import tpu_sc as plsc

NUM_CLASSES = 1000
FEAT_DIM = 128
BATCH = 16384
ALPHA = 0.99

PAD_CLASSES = 1024
NC = 2
NS = 16
NWT = NC * NS
ROWS_PER_W = BATCH // NWT
CHUNK = 128
NCHUNK = ROWS_PER_W // CHUNK
CLS_PER_S = PAD_CLASSES // NS
LANES = 16
VL = FEAT_DIM // LANES


def _scatter_body(feat_hbm, lbl_hbm, psum_hbm, pcnt_hbm,
                  lbl_v, feat_v, ones_v, zero_v,
                  shared_acc, shared_cnt, sem0, sem1, sem2, sem3, sem_sc):
    cid = lax.axis_index("c")
    sid = lax.axis_index("s")
    wid = cid * NS + sid
    cls_base = sid * CLS_PER_S
    zeros16 = jnp.zeros((LANES,), jnp.float32)
    ones16 = jnp.ones((LANES,), jnp.float32)

    def zero_row(r, _):
        for j in range(VL):
            zero_v[r, pl.ds(j * LANES, LANES)] = zeros16
        return _
    lax.fori_loop(0, CLS_PER_S, zero_row, None)

    pltpu.sync_copy(zero_v, shared_acc.at[pl.ds(cls_base, CLS_PER_S)])
    pltpu.sync_copy(zero_v, shared_cnt.at[pl.ds(cls_base, CLS_PER_S)])
    plsc.subcore_barrier()

    pltpu.sync_copy(lbl_hbm.at[wid], lbl_v)
    in_sems = [sem0, sem1, sem2, sem3]
    copies = []
    for j in range(NCHUNK):
        copies.append(pltpu.async_copy(
            feat_hbm.at[pl.ds(wid * ROWS_PER_W + j * CHUNK, CHUNK)],
            feat_v.at[j], in_sems[j]))

    def ones_row(r, _):
        for j in range(VL):
            ones_v[r, pl.ds(j * LANES, LANES)] = ones16
        return _
    lax.fori_loop(0, CHUNK, ones_row, None)

    scatters = []
    for j in range(NCHUNK):
        scatters.append(pltpu.async_copy(
            ones_v, shared_cnt.at[lbl_v.at[j]], sem_sc, add=True))
    for j in range(NCHUNK):
        copies[j].wait()
        scatters.append(pltpu.async_copy(
            feat_v.at[j], shared_acc.at[lbl_v.at[j]], sem_sc, add=True))
    for d in scatters:
        d.wait()
    plsc.subcore_barrier()

    w1 = pltpu.async_copy(shared_acc.at[pl.ds(cls_base, CLS_PER_S)],
                          psum_hbm.at[cid, pl.ds(cls_base, CLS_PER_S)], sem0)
    w2 = pltpu.async_copy(shared_cnt.at[pl.ds(cls_base, CLS_PER_S)],
                          pcnt_hbm.at[cid, pl.ds(cls_base, CLS_PER_S)], sem1)
    w1.wait()
    w2.wait()


def _ema_body(psum_ref, pcnt_ref, proto_ref, out_ref):
    s = psum_ref[0, :NUM_CLASSES, :] + psum_ref[1, :NUM_CLASSES, :]
    c = pcnt_ref[0, :NUM_CLASSES, :] + pcnt_ref[1, :NUM_CLASSES, :]
    p = proto_ref[...]
    mean = s / jnp.maximum(c, 1.0)
    out_ref[...] = jnp.where(c > 0.0, ALPHA * p + (1.0 - ALPHA) * mean, p)


@jax.jit
def _run(features, labels3, prototypes):
    mesh = plsc.VectorSubcoreMesh(
        core_axis_name="c", subcore_axis_name="s", num_cores=NC,
        num_subcores=NS)
    psum, pcnt = pl.kernel(
        _scatter_body,
        out_type=(
            jax.ShapeDtypeStruct((NC, PAD_CLASSES, FEAT_DIM), jnp.float32),
            jax.ShapeDtypeStruct((NC, PAD_CLASSES, FEAT_DIM), jnp.float32)),
        mesh=mesh,
        scratch_types=[
            pltpu.VMEM((NCHUNK, CHUNK), jnp.int32),
            pltpu.VMEM((NCHUNK, CHUNK, FEAT_DIM), jnp.float32),
            pltpu.VMEM((CHUNK, FEAT_DIM), jnp.float32),
            pltpu.VMEM((CLS_PER_S, FEAT_DIM), jnp.float32),
            pltpu.VMEM_SHARED((PAD_CLASSES, FEAT_DIM), jnp.float32),
            pltpu.VMEM_SHARED((PAD_CLASSES, FEAT_DIM), jnp.float32),
            pltpu.SemaphoreType.DMA,
            pltpu.SemaphoreType.DMA,
            pltpu.SemaphoreType.DMA,
            pltpu.SemaphoreType.DMA,
            pltpu.SemaphoreType.DMA,
        ],
    )(features, labels3)

    out = pl.pallas_call(
        _ema_body,
        out_shape=jax.ShapeDtypeStruct((NUM_CLASSES, FEAT_DIM), jnp.float32),
    )(psum, pcnt, prototypes)
    return out


def kernel(features, labels, prototypes):
    labels3 = labels.astype(jnp.int32).reshape(NWT, NCHUNK, CHUNK)
    return _run(features, labels3, prototypes)


# speedup vs baseline: 1.1387x; 1.0374x over previous
import jax
import jax.numpy as jnp
from jax import lax
from jax.experimental import pallas as pl
from jax.experimental.pallas import tpu as pltpu
from jax.experimental.pallas import tpu_sc as plsc

NUM_CLASSES = 1000
FEAT_DIM = 128
BATCH = 16384
ALPHA = 0.99

PAD_CLASSES = 1024
NC = 2
NS = 16
NWT = NC * NS
ROWS_PER_W = BATCH // NWT
CHUNK = 128
NCHUNK = ROWS_PER_W // CHUNK
CLS_PER_S = PAD_CLASSES // NS
LANES = 16
VL = FEAT_DIM // LANES


def _scatter_body(feat_hbm, lbl_hbm, psum_hbm, pcnt_hbm,
                  lbl_v, feat_v, ones_v, zero_v,
                  shared_acc, shared_cnt,
                  sem0, sem1, sem2, sem3, sem4, sem5, sem6, sem_sc):
    cid = lax.axis_index("c")
    sid = lax.axis_index("s")
    wid = cid * NS + sid
    cls_base = sid * CLS_PER_S
    zeros16 = jnp.zeros((LANES,), jnp.float32)
    ones16 = jnp.ones((LANES,), jnp.float32)

    lbl_cp = pltpu.async_copy(lbl_hbm.at[wid], lbl_v, sem4)
    in_sems = [sem0, sem1, sem2, sem3]
    copies = []
    for j in range(NCHUNK):
        copies.append(pltpu.async_copy(
            feat_hbm.at[pl.ds(wid * ROWS_PER_W + j * CHUNK, CHUNK)],
            feat_v.at[j], in_sems[j]))

    def zero_row(r, _):
        for j in range(VL):
            zero_v[r, pl.ds(j * LANES, LANES)] = zeros16
        return _
    lax.fori_loop(0, CLS_PER_S, zero_row, None)

    z1 = pltpu.async_copy(zero_v, shared_acc.at[pl.ds(cls_base, CLS_PER_S)],
                          sem5)
    z2 = pltpu.async_copy(zero_v, shared_cnt.at[pl.ds(cls_base, CLS_PER_S)],
                          sem6)

    def ones_row(r, _):
        for j in range(VL):
            ones_v[r, pl.ds(j * LANES, LANES)] = ones16
        return _
    lax.fori_loop(0, CHUNK, ones_row, None)

    z1.wait()
    z2.wait()
    plsc.subcore_barrier()

    lbl_cp.wait()
    scatters = []
    for j in range(NCHUNK):
        scatters.append(pltpu.async_copy(
            ones_v, shared_cnt.at[lbl_v.at[j]], sem_sc, add=True))
    for j in range(NCHUNK):
        copies[j].wait()
        scatters.append(pltpu.async_copy(
            feat_v.at[j], shared_acc.at[lbl_v.at[j]], sem_sc, add=True))
    for d in scatters:
        d.wait()
    plsc.subcore_barrier()

    w1 = pltpu.async_copy(shared_acc.at[pl.ds(cls_base, CLS_PER_S)],
                          psum_hbm.at[cid, pl.ds(cls_base, CLS_PER_S)], sem0)
    w2 = pltpu.async_copy(shared_cnt.at[pl.ds(cls_base, CLS_PER_S)],
                          pcnt_hbm.at[cid, pl.ds(cls_base, CLS_PER_S)], sem1)
    w1.wait()
    w2.wait()


def _ema_body(psum_ref, pcnt_ref, proto_ref, out_ref):
    s = psum_ref[0, :NUM_CLASSES, :] + psum_ref[1, :NUM_CLASSES, :]
    c16 = pcnt_ref[0, :NUM_CLASSES, :] + pcnt_ref[1, :NUM_CLASSES, :]
    c = jnp.broadcast_to(c16[:, 0:1], (NUM_CLASSES, FEAT_DIM))
    p = proto_ref[...]
    mean = s / jnp.maximum(c, 1.0)
    out_ref[...] = jnp.where(c > 0.0, ALPHA * p + (1.0 - ALPHA) * mean, p)


@jax.jit
def _run(features, labels3, prototypes):
    mesh = plsc.VectorSubcoreMesh(
        core_axis_name="c", subcore_axis_name="s", num_cores=NC,
        num_subcores=NS)
    psum, pcnt = pl.kernel(
        _scatter_body,
        out_type=(
            jax.ShapeDtypeStruct((NC, PAD_CLASSES, FEAT_DIM), jnp.float32),
            jax.ShapeDtypeStruct((NC, PAD_CLASSES, FEAT_DIM), jnp.float32)),
        mesh=mesh,
        scratch_types=[
            pltpu.VMEM((NCHUNK, CHUNK), jnp.int32),
            pltpu.VMEM((NCHUNK, CHUNK, FEAT_DIM), jnp.float32),
            pltpu.VMEM((CHUNK, FEAT_DIM), jnp.float32),
            pltpu.VMEM((CLS_PER_S, FEAT_DIM), jnp.float32),
            pltpu.VMEM_SHARED((PAD_CLASSES, FEAT_DIM), jnp.float32),
            pltpu.VMEM_SHARED((PAD_CLASSES, FEAT_DIM), jnp.float32),
            pltpu.SemaphoreType.DMA,
            pltpu.SemaphoreType.DMA,
            pltpu.SemaphoreType.DMA,
            pltpu.SemaphoreType.DMA,
            pltpu.SemaphoreType.DMA,
            pltpu.SemaphoreType.DMA,
            pltpu.SemaphoreType.DMA,
            pltpu.SemaphoreType.DMA,
        ],
    )(features, labels3)

    out = pl.pallas_call(
        _ema_body,
        out_shape=jax.ShapeDtypeStruct((NUM_CLASSES, FEAT_DIM), jnp.float32),
    )(psum, pcnt, prototypes)
    return out


def kernel(features, labels, prototypes):
    labels3 = labels.astype(jnp.int32).reshape(NWT, NCHUNK, CHUNK)
    return _run(features, labels3, prototypes)


# speedup vs baseline: 1.5818x; 1.3892x over previous
import jax
import jax.numpy as jnp
from jax import lax
from jax.experimental import pallas as pl
from jax.experimental.pallas import tpu as pltpu
from jax.experimental.pallas import tpu_sc as plsc

NUM_CLASSES = 1000
FEAT_DIM = 128
BATCH = 16384
ALPHA = 0.99

PAD_CLASSES = 1024
NC = 2
NS = 16
NWT = NC * NS
ROWS_PER_W = BATCH // NWT
CHUNK = 128
NCHUNK = ROWS_PER_W // CHUNK
CLS_PER_S = PAD_CLASSES // NS
LANES = 16
VL = FEAT_DIM // LANES


def _scatter_body(feat_hbm, lbl_hbm, zeros_hbm, ones_hbm, psum_hbm, pcnt_hbm,
                  lbl_v, feat_v, ones_v,
                  shared_acc, shared_cnt,
                  sem0, sem1, sem2, sem3, sem4, sem5, sem6, sem_sc):
    cid = lax.axis_index("c")
    sid = lax.axis_index("s")
    wid = cid * NS + sid
    cls_base = sid * CLS_PER_S

    lbl_cp = pltpu.async_copy(lbl_hbm.at[wid], lbl_v, sem4)
    ones_cp = pltpu.async_copy(ones_hbm, ones_v, sem5)
    z1 = pltpu.async_copy(zeros_hbm,
                          shared_acc.at[pl.ds(cls_base, CLS_PER_S)], sem6)
    z2 = pltpu.async_copy(zeros_hbm,
                          shared_cnt.at[pl.ds(cls_base, CLS_PER_S)], sem_sc)
    in_sems = [sem0, sem1, sem2, sem3]
    copies = []
    for j in range(NCHUNK):
        copies.append(pltpu.async_copy(
            feat_hbm.at[pl.ds(wid * ROWS_PER_W + j * CHUNK, CHUNK)],
            feat_v.at[j], in_sems[j]))

    z1.wait()
    z2.wait()
    plsc.subcore_barrier()

    lbl_cp.wait()
    ones_cp.wait()
    scatters = []
    for j in range(NCHUNK):
        scatters.append(pltpu.async_copy(
            ones_v, shared_cnt.at[lbl_v.at[j]], sem_sc, add=True))
    for j in range(NCHUNK):
        copies[j].wait()
        scatters.append(pltpu.async_copy(
            feat_v.at[j], shared_acc.at[lbl_v.at[j]], sem_sc, add=True))
    for d in scatters:
        d.wait()
    plsc.subcore_barrier()

    w1 = pltpu.async_copy(shared_acc.at[pl.ds(cls_base, CLS_PER_S)],
                          psum_hbm.at[cid, pl.ds(cls_base, CLS_PER_S)], sem0)
    w2 = pltpu.async_copy(shared_cnt.at[pl.ds(cls_base, CLS_PER_S)],
                          pcnt_hbm.at[cid, pl.ds(cls_base, CLS_PER_S)], sem1)
    w1.wait()
    w2.wait()


def _ema_body(psum_ref, pcnt_ref, proto_ref, out_ref):
    s = psum_ref[0, :NUM_CLASSES, :] + psum_ref[1, :NUM_CLASSES, :]
    c16 = pcnt_ref[0, :NUM_CLASSES, :] + pcnt_ref[1, :NUM_CLASSES, :]
    c = jnp.broadcast_to(c16[:, 0:1], (NUM_CLASSES, FEAT_DIM))
    p = proto_ref[...]
    mean = s / jnp.maximum(c, 1.0)
    out_ref[...] = jnp.where(c > 0.0, ALPHA * p + (1.0 - ALPHA) * mean, p)


@jax.jit
def _run(features, labels3, prototypes):
    mesh = plsc.VectorSubcoreMesh(
        core_axis_name="c", subcore_axis_name="s", num_cores=NC,
        num_subcores=NS)
    psum, pcnt = pl.kernel(
        _scatter_body,
        out_type=(
            jax.ShapeDtypeStruct((NC, PAD_CLASSES, FEAT_DIM), jnp.float32),
            jax.ShapeDtypeStruct((NC, PAD_CLASSES, FEAT_DIM), jnp.float32)),
        mesh=mesh,
        scratch_types=[
            pltpu.VMEM((NCHUNK, CHUNK), jnp.int32),
            pltpu.VMEM((NCHUNK, CHUNK, FEAT_DIM), jnp.float32),
            pltpu.VMEM((CHUNK, FEAT_DIM), jnp.float32),
            pltpu.VMEM_SHARED((PAD_CLASSES, FEAT_DIM), jnp.float32),
            pltpu.VMEM_SHARED((PAD_CLASSES, FEAT_DIM), jnp.float32),
            pltpu.SemaphoreType.DMA,
            pltpu.SemaphoreType.DMA,
            pltpu.SemaphoreType.DMA,
            pltpu.SemaphoreType.DMA,
            pltpu.SemaphoreType.DMA,
            pltpu.SemaphoreType.DMA,
            pltpu.SemaphoreType.DMA,
            pltpu.SemaphoreType.DMA,
        ],
    )(features, labels3,
      jnp.zeros((CLS_PER_S, FEAT_DIM), jnp.float32),
      jnp.ones((CHUNK, FEAT_DIM), jnp.float32))

    out = pl.pallas_call(
        _ema_body,
        out_shape=jax.ShapeDtypeStruct((NUM_CLASSES, FEAT_DIM), jnp.float32),
    )(psum, pcnt, prototypes)
    return out


def kernel(features, labels, prototypes):
    labels3 = labels.astype(jnp.int32).reshape(NWT, NCHUNK, CHUNK)
    return _run(features, labels3, prototypes)
